# baseline (speedup 1.0000x reference)
import functools

import jax
import jax.numpy as jnp
from jax.experimental import pallas as pl
import jax.experimental.pallas.tpu as pltpu

CHUNK = 8192
NCORE = 2


def _sum_kernel(x_ref, w_ref, b_ref, part_ref, acc_ref, *, inner):
    i = pl.program_id(1)
    lt = jax.lax.dot_general(
        w_ref[...].astype(jnp.bfloat16), x_ref[...].astype(jnp.bfloat16),
        (((1,), (1,)), ((), ())),
        preferred_element_type=jnp.float32,
    ) + b_ref[...]
    m = jnp.max(lt, axis=0, keepdims=True)
    e = jnp.exp(lt - m)
    s = jnp.sum(e, axis=0, keepdims=True)
    p = e / s
    part = jnp.sum(p, axis=1, keepdims=True)

    @pl.when(i == 0)
    def _init():
        acc_ref[...] = part

    @pl.when(i > 0)
    def _acc():
        acc_ref[...] += part

    @pl.when(i == inner - 1)
    def _flush():
        part_ref[0, :, :] = acc_ref[...]


def _finish_kernel(part_ref, ssm_ref, msm_ref, out_ref, *, inv_n):
    avg = jnp.sum(part_ref[...], axis=0) * inv_n
    nssm = ssm_ref.shape[0]
    out_ref[0:nssm, :] = ssm_ref[...] * avg[0:nssm, :]
    out_ref[nssm:, :] = msm_ref[...] * avg[nssm:, :]


def kernel(input_stream, ssm_slots, msm_slots, router_w, router_b,
           compress_w, compress_b):
    del compress_w, compress_b
    n, d = input_stream.shape
    nslots = router_w.shape[0]
    inner = n // (CHUNK * NCORE)

    parts = pl.pallas_call(
        functools.partial(_sum_kernel, inner=inner),
        grid=(NCORE, inner),
        in_specs=[
            pl.BlockSpec((CHUNK, d), lambda c, i: (c * inner + i, 0)),
            pl.BlockSpec((nslots, d), lambda c, i: (0, 0)),
            pl.BlockSpec((nslots, 1), lambda c, i: (0, 0)),
        ],
        out_specs=pl.BlockSpec((1, nslots, 1), lambda c, i: (c, 0, 0)),
        out_shape=jax.ShapeDtypeStruct((NCORE, nslots, 1), jnp.float32),
        scratch_shapes=[pltpu.VMEM((nslots, 1), jnp.float32)],
        compiler_params=pltpu.CompilerParams(
            dimension_semantics=("parallel", "arbitrary")),
    )(input_stream, router_w, router_b.reshape(nslots, 1))

    out = pl.pallas_call(
        functools.partial(_finish_kernel, inv_n=1.0 / n),
        out_shape=jax.ShapeDtypeStruct((nslots, d), jnp.float32),
    )(parts, ssm_slots, msm_slots)
    return out


# speedup vs baseline: 1.0493x; 1.0493x over previous
import functools

import jax
import jax.numpy as jnp
from jax.experimental import pallas as pl
import jax.experimental.pallas.tpu as pltpu

CHUNK = 8192


def _router_kernel(x_ref, w_ref, b_ref, ssm_ref, msm_ref, out_ref, acc_ref,
                   *, grid, inv_n):
    i = pl.program_id(0)
    lt = jax.lax.dot_general(
        w_ref[...].astype(jnp.bfloat16), x_ref[...].astype(jnp.bfloat16),
        (((1,), (1,)), ((), ())),
        preferred_element_type=jnp.float32,
    ) + b_ref[...].T
    m = jnp.max(lt, axis=0, keepdims=True)
    e = jnp.exp(lt - m)
    s = jnp.sum(e, axis=0, keepdims=True)
    p = e / s
    part = jnp.sum(p, axis=1, keepdims=True)

    @pl.when(i == 0)
    def _init():
        acc_ref[...] = part

    @pl.when(i > 0)
    def _acc():
        acc_ref[...] += part

    @pl.when(i == grid - 1)
    def _finish():
        nssm = ssm_ref.shape[0]
        avg = acc_ref[...] * inv_n
        out_ref[0:nssm, :] = ssm_ref[...] * avg[0:nssm, :]
        out_ref[nssm:, :] = msm_ref[...] * avg[nssm:, :]


def kernel(input_stream, ssm_slots, msm_slots, router_w, router_b,
           compress_w, compress_b):
    del compress_w, compress_b
    n, d = input_stream.shape
    nslots = router_w.shape[0]
    grid = n // CHUNK

    out = pl.pallas_call(
        functools.partial(_router_kernel, grid=grid, inv_n=1.0 / n),
        grid=(grid,),
        in_specs=[
            pl.BlockSpec((CHUNK, d), lambda i: (i, 0)),
            pl.BlockSpec((nslots, d), lambda i: (0, 0)),
            pl.BlockSpec((1, nslots), lambda i: (0, 0)),
            pl.BlockSpec(ssm_slots.shape, lambda i: (0, 0)),
            pl.BlockSpec(msm_slots.shape, lambda i: (0, 0)),
        ],
        out_specs=pl.BlockSpec((nslots, d), lambda i: (0, 0)),
        out_shape=jax.ShapeDtypeStruct((nslots, d), jnp.float32),
        scratch_shapes=[pltpu.VMEM((nslots, 1), jnp.float32)],
    )(input_stream, router_w, router_b.reshape(1, nslots),
      ssm_slots, msm_slots)
    return out
